# baseline (speedup 1.0000x reference)
import functools

import jax
import jax.numpy as jnp
from jax import lax
from jax.experimental import pallas as pl
from jax.experimental.pallas import tpu as pltpu
from jax.experimental.pallas import tpu_sc as plsc

B = 16384
L = 200
D = 32
NC, NS = 2, 16
NW = NC * NS
RPW = B // NW
R = 8
TOK = R * L
NCHUNK = RPW // R
GSZ = 128

_BURSTS = []
_o = 0
while _o < TOK:
    _BURSTS.append((_o, min(GSZ, TOK - _o)))
    _o += _BURSTS[-1][1]


def _sc_body(idx_hbm, table_hbm, out_hbm, idx_v, gat_v, out_v, row0_v,
             sem_g0, sem_g1, sem_i0, sem_i1):
    wid = lax.axis_index("s") * NC + lax.axis_index("c")
    base_tok = wid * (RPW * L)
    sem_g = (sem_g0, sem_g1)
    sem_i = (sem_i0, sem_i1)

    pltpu.sync_copy(table_hbm.at[pl.ds(0, 1)], row0_v)
    row0a = row0_v[0, pl.ds(0, 16)]
    row0b = row0_v[0, pl.ds(16, 16)]
    lane = lax.iota(jnp.int32, 16)
    lane_lo = lane < 8

    def fire_idx(g, slot):
        off = pl.multiple_of(base_tok + g * TOK, 8)
        pltpu.async_copy(idx_hbm.at[pl.ds(off, TOK)], idx_v.at[slot],
                         sem_i[slot])

    def wait_idx(g, slot):
        off = pl.multiple_of(base_tok + g * TOK, 8)
        pltpu.make_async_copy(idx_hbm.at[pl.ds(off, TOK)], idx_v.at[slot],
                              sem_i[slot]).wait()

    def fire_gather(slot):
        for (bo, bs) in _BURSTS:
            pltpu.async_copy(
                table_hbm.at[idx_v.at[slot, pl.ds(bo, bs)]],
                gat_v.at[slot, pl.ds(bo, bs)],
                sem_g[slot])

    def wait_gather(slot):
        for (bo, bs) in _BURSTS:
            pltpu.make_async_copy(
                table_hbm.at[idx_v.at[slot, pl.ds(bo, bs)]],
                gat_v.at[slot, pl.ds(bo, bs)],
                sem_g[slot]).wait()

    def zero_counts(slot):
        cnts = []
        for p in range(R // 2):
            pbase = p * 2 * L
            cnt_a = jnp.zeros((16,), jnp.int32)
            cnt_b = jnp.zeros((16,), jnp.int32)
            for k in range(2 * L // 16):
                v = idx_v[slot, pl.ds(pbase + k * 16, 16)] == 0
                if k < 12:
                    cnt_a = cnt_a + plsc.all_reduce_population_count(v)
                elif k == 12:
                    cnt_a = cnt_a + plsc.all_reduce_population_count(v & lane_lo)
                    cnt_b = cnt_b + plsc.all_reduce_population_count(
                        v & (~lane_lo))
                else:
                    cnt_b = cnt_b + plsc.all_reduce_population_count(v)
            cnts.append(cnt_a)
            cnts.append(cnt_b)
        return cnts

    def reduce_chunk(g, slot):
        cnts = zero_counts(slot)
        for r in range(R):
            row = g * R + r
            tbase = r * L
            zero = jnp.zeros((16,), jnp.float32)

            def body(j, accs, tbase=tbase, slot=slot):
                a0, a1, b0, b1, c0, c1, d0, d1 = accs
                t = tbase + j * 4
                a0 = a0 + gat_v[slot, t, pl.ds(0, 16)]
                a1 = a1 + gat_v[slot, t, pl.ds(16, 16)]
                b0 = b0 + gat_v[slot, t + 1, pl.ds(0, 16)]
                b1 = b1 + gat_v[slot, t + 1, pl.ds(16, 16)]
                c0 = c0 + gat_v[slot, t + 2, pl.ds(0, 16)]
                c1 = c1 + gat_v[slot, t + 2, pl.ds(16, 16)]
                d0 = d0 + gat_v[slot, t + 3, pl.ds(0, 16)]
                d1 = d1 + gat_v[slot, t + 3, pl.ds(16, 16)]
                return (a0, a1, b0, b1, c0, c1, d0, d1)

            accs = lax.fori_loop(0, L // 4, body, (zero,) * 8)
            s0 = (accs[0] + accs[2]) + (accs[4] + accs[6])
            s1 = (accs[1] + accs[3]) + (accs[5] + accs[7])
            cntf = cnts[r].astype(jnp.float32)
            ln = float(L) - cntf
            inv = jnp.where(ln > 0.0, 1.0 / (ln + 1e-8), jnp.zeros_like(ln))
            out_v[row, pl.ds(0, 16)] = (s0 - cntf * row0a) * inv
            out_v[row, pl.ds(16, 16)] = (s1 - cntf * row0b) * inv

    fire_idx(0, 0)
    wait_idx(0, 0)
    fire_gather(0)
    fire_idx(1, 1)

    def pipe_body(m, carry):
        a = 2 * m
        b = a + 1
        wait_idx(b, 1)
        fire_gather(1)
        wait_gather(0)

        @pl.when(a + 2 < NCHUNK)
        def _():
            fire_idx(a + 2, 0)


        @pl.when(a + 2 < NCHUNK)
        def _():
            wait_idx(a + 2, 0)
            fire_gather(0)

        wait_gather(1)

        @pl.when(b + 2 < NCHUNK)
        def _():
            fire_idx(b + 2, 1)

        return carry

    lax.fori_loop(0, NCHUNK // 2, pipe_body, 0)
    orow = pl.multiple_of(wid * RPW, 8)
    pltpu.sync_copy(out_v, out_hbm.at[pl.ds(orow, RPW)])


_avg_embed_sc = functools.partial(
    pl.kernel,
    out_type=jax.ShapeDtypeStruct((B, D), jnp.float32),
    mesh=plsc.VectorSubcoreMesh(
        core_axis_name="c", subcore_axis_name="s",
        num_cores=NC, num_subcores=NS),
    compiler_params=pltpu.CompilerParams(
        needs_layout_passes=False, use_tc_tiling_on_sc=False),
    scratch_types=[
        pltpu.VMEM((2, TOK), jnp.int32),
        pltpu.VMEM((2, TOK, D), jnp.float32),
        pltpu.VMEM((RPW, D), jnp.float32),
        pltpu.VMEM((1, D), jnp.float32),
        pltpu.SemaphoreType.DMA,
        pltpu.SemaphoreType.DMA,
        pltpu.SemaphoreType.DMA,
        pltpu.SemaphoreType.DMA,
    ],
)(_sc_body)


def kernel(inputs, embeddings):
    return _avg_embed_sc(inputs.reshape(-1), embeddings)


# speedup vs baseline: 1.0179x; 1.0179x over previous
import functools

import jax
import jax.numpy as jnp
from jax import lax
from jax.experimental import pallas as pl
from jax.experimental.pallas import tpu as pltpu
from jax.experimental.pallas import tpu_sc as plsc

B = 16384
L = 200
D = 32
NC, NS = 2, 16
NW = NC * NS
RPW = B // NW
R = 8
TOK = R * L
NCHUNK = RPW // R
GSZ = 1600

_BURSTS = []
_o = 0
while _o < TOK:
    _BURSTS.append((_o, min(GSZ, TOK - _o)))
    _o += _BURSTS[-1][1]


def _sc_body(idx_hbm, table_hbm, out_hbm, idx_v, gat_v, out_v, row0_v,
             sem_g0, sem_g1, sem_i0, sem_i1):
    wid = lax.axis_index("s") * NC + lax.axis_index("c")
    base_tok = wid * (RPW * L)
    sem_g = (sem_g0, sem_g1)
    sem_i = (sem_i0, sem_i1)

    pltpu.sync_copy(table_hbm.at[pl.ds(0, 1)], row0_v)
    row0a = row0_v[0, pl.ds(0, 16)]
    row0b = row0_v[0, pl.ds(16, 16)]
    lane = lax.iota(jnp.int32, 16)
    lane_lo = lane < 8

    def fire_idx(g, slot):
        off = pl.multiple_of(base_tok + g * TOK, 8)
        pltpu.async_copy(idx_hbm.at[pl.ds(off, TOK)], idx_v.at[slot],
                         sem_i[slot])

    def wait_idx(g, slot):
        off = pl.multiple_of(base_tok + g * TOK, 8)
        pltpu.make_async_copy(idx_hbm.at[pl.ds(off, TOK)], idx_v.at[slot],
                              sem_i[slot]).wait()

    def fire_gather(slot):
        for (bo, bs) in _BURSTS:
            pltpu.async_copy(
                table_hbm.at[idx_v.at[slot, pl.ds(bo, bs)]],
                gat_v.at[slot, pl.ds(bo, bs)],
                sem_g[slot])

    def wait_gather(slot):
        for (bo, bs) in _BURSTS:
            pltpu.make_async_copy(
                table_hbm.at[idx_v.at[slot, pl.ds(bo, bs)]],
                gat_v.at[slot, pl.ds(bo, bs)],
                sem_g[slot]).wait()

    def zero_counts(slot):
        cnts = []
        for p in range(R // 2):
            pbase = p * 2 * L
            cnt_a = jnp.zeros((16,), jnp.int32)
            cnt_b = jnp.zeros((16,), jnp.int32)
            for k in range(2 * L // 16):
                v = idx_v[slot, pl.ds(pbase + k * 16, 16)] == 0
                if k < 12:
                    cnt_a = cnt_a + plsc.all_reduce_population_count(v)
                elif k == 12:
                    cnt_a = cnt_a + plsc.all_reduce_population_count(v & lane_lo)
                    cnt_b = cnt_b + plsc.all_reduce_population_count(
                        v & (~lane_lo))
                else:
                    cnt_b = cnt_b + plsc.all_reduce_population_count(v)
            cnts.append(cnt_a)
            cnts.append(cnt_b)
        return cnts

    def reduce_chunk(g, slot):
        cnts = zero_counts(slot)
        for r in range(R):
            row = g * R + r
            tbase = r * L
            zero = jnp.zeros((16,), jnp.float32)

            def body(j, accs, tbase=tbase, slot=slot):
                a0, a1, b0, b1, c0, c1, d0, d1 = accs
                t = tbase + j * 4
                a0 = a0 + gat_v[slot, t, pl.ds(0, 16)]
                a1 = a1 + gat_v[slot, t, pl.ds(16, 16)]
                b0 = b0 + gat_v[slot, t + 1, pl.ds(0, 16)]
                b1 = b1 + gat_v[slot, t + 1, pl.ds(16, 16)]
                c0 = c0 + gat_v[slot, t + 2, pl.ds(0, 16)]
                c1 = c1 + gat_v[slot, t + 2, pl.ds(16, 16)]
                d0 = d0 + gat_v[slot, t + 3, pl.ds(0, 16)]
                d1 = d1 + gat_v[slot, t + 3, pl.ds(16, 16)]
                return (a0, a1, b0, b1, c0, c1, d0, d1)

            accs = lax.fori_loop(0, L // 4, body, (zero,) * 8)
            s0 = (accs[0] + accs[2]) + (accs[4] + accs[6])
            s1 = (accs[1] + accs[3]) + (accs[5] + accs[7])
            cntf = cnts[r].astype(jnp.float32)
            ln = float(L) - cntf
            inv = jnp.where(ln > 0.0, 1.0 / (ln + 1e-8), jnp.zeros_like(ln))
            out_v[row, pl.ds(0, 16)] = (s0 - cntf * row0a) * inv
            out_v[row, pl.ds(16, 16)] = (s1 - cntf * row0b) * inv

    fire_idx(0, 0)
    wait_idx(0, 0)
    fire_gather(0)
    fire_idx(1, 1)

    def pipe_body(m, carry):
        a = 2 * m
        b = a + 1
        wait_idx(b, 1)
        fire_gather(1)
        wait_gather(0)

        @pl.when(a + 2 < NCHUNK)
        def _():
            fire_idx(a + 2, 0)

        reduce_chunk(a, 0)

        @pl.when(a + 2 < NCHUNK)
        def _():
            wait_idx(a + 2, 0)
            fire_gather(0)

        wait_gather(1)

        @pl.when(b + 2 < NCHUNK)
        def _():
            fire_idx(b + 2, 1)

        reduce_chunk(b, 1)
        return carry

    lax.fori_loop(0, NCHUNK // 2, pipe_body, 0)
    orow = pl.multiple_of(wid * RPW, 8)
    pltpu.sync_copy(out_v, out_hbm.at[pl.ds(orow, RPW)])


_avg_embed_sc = functools.partial(
    pl.kernel,
    out_type=jax.ShapeDtypeStruct((B, D), jnp.float32),
    mesh=plsc.VectorSubcoreMesh(
        core_axis_name="c", subcore_axis_name="s",
        num_cores=NC, num_subcores=NS),
    compiler_params=pltpu.CompilerParams(
        needs_layout_passes=False, use_tc_tiling_on_sc=False),
    scratch_types=[
        pltpu.VMEM((2, TOK), jnp.int32),
        pltpu.VMEM((2, TOK, D), jnp.float32),
        pltpu.VMEM((RPW, D), jnp.float32),
        pltpu.VMEM((1, D), jnp.float32),
        pltpu.SemaphoreType.DMA,
        pltpu.SemaphoreType.DMA,
        pltpu.SemaphoreType.DMA,
        pltpu.SemaphoreType.DMA,
    ],
)(_sc_body)


def kernel(inputs, embeddings):
    return _avg_embed_sc(inputs.reshape(-1), embeddings)
